# baseline (speedup 1.0000x reference)
import functools

import jax
import jax.numpy as jnp
from jax import lax
from jax.experimental import pallas as pl
from jax.experimental.pallas import tpu as pltpu
from jax.experimental.pallas import tpu_sc as plsc

N = 10000
E = 320000
DELTA = 2.0

NC = 2
NS = 16
NW = NC * NS
EPT = E // NW
K = 125
NCHUNK = EPT // K
NBUF = 4
NN = 10240
NPS = NN // NS
ZR = 80

R4 = NN * 32 // 128
R8 = NN * 16 // 128

_mesh = plsc.VectorSubcoreMesh(core_axis_name="c", subcore_axis_name="s")
_sc_params = pltpu.CompilerParams(use_tc_tiling_on_sc=False)


def _zero_fill(ref, rows, width):
    def body(i, _):
        for j in range(width // 16):
            ref[i, pl.ds(j * 16, 16)] = jnp.zeros((16,), jnp.float32)
        return 0
    lax.fori_loop(0, rows, body, 0)


def _zero_shared_slice(acc, zq, row0):
    for r in range(NPS // ZR):
        pltpu.sync_copy(zq, acc.at[pl.ds(row0 + r * ZR, ZR)])


def _make_msg_kernel(width):
    @functools.partial(
        pl.kernel,
        out_type=jax.ShapeDtypeStruct((NC * NN, width), jnp.float32),
        mesh=_mesh,
        compiler_params=_sc_params,
        scratch_types=[
            pltpu.VMEM((NCHUNK, K), jnp.int32),
            pltpu.VMEM((NCHUNK, K), jnp.int32),
            [pltpu.VMEM((K, width), jnp.float32)] * NBUF,
            pltpu.VMEM((ZR, width), jnp.float32),
            pltpu.VMEM((NPS, width), jnp.float32),
            pltpu.VMEM((NPS // 128, 128), jnp.int32),
            pltpu.VMEM_SHARED((NN, width), jnp.float32),
            pltpu.VMEM_SHARED((NN, width), jnp.float32),
            [pltpu.SemaphoreType.DMA] * NBUF,
            [pltpu.SemaphoreType.DMA] * 2,
        ],
    )
    def msg_kernel(g_hbm, eidx_hbm, out_hbm, src_v, dst_v, bufs, zq, tbuf,
                   iidx, accA, accB, gsems, ssems):
        c = lax.axis_index("c")
        s = lax.axis_index("s")
        wid = c * NS + s
        pltpu.sync_copy(eidx_hbm.at[wid], src_v)
        pltpu.sync_copy(eidx_hbm.at[NW + wid], dst_v)
        _zero_fill(zq, ZR, width)
        _zero_shared_slice(accA, zq, s * NPS)
        _zero_shared_slice(accB, zq, s * NPS)
        ii16 = lax.iota(jnp.int32, 16)
        base_row = s * NPS
        for kk in range(NPS // 128):
            for j in range(8):
                iidx[kk, pl.ds(16 * j, 16)] = base_row + kk * 128 + j * 16 + ii16
        plsc.subcore_barrier()

        accs = [accA, accB]
        NLOOP = NCHUNK // NBUF
        for b in range(2):
            pltpu.async_copy(g_hbm.at[src_v.at[b]], bufs[b], gsems[b])

        def body(j, _):
            for b in range(NBUF):
                ch = j * NBUF + b
                p = b % 2
                pltpu.make_async_copy(g_hbm.at[src_v.at[ch]], bufs[b],
                                      gsems[b]).wait()
                if b >= 2:
                    pltpu.make_async_copy(bufs[b - 2],
                                          accs[p].at[dst_v.at[ch - 2]],
                                          ssems[p]).wait()
                    @pl.when(j < NLOOP - 1)
                    def _():
                        pltpu.async_copy(g_hbm.at[src_v.at[ch + 2]],
                                         bufs[b - 2], gsems[b - 2])
                else:
                    @pl.when(j > 0)
                    def _():
                        pltpu.make_async_copy(bufs[b + 2],
                                              accs[p].at[dst_v.at[ch - 2]],
                                              ssems[p]).wait()
                    pltpu.async_copy(g_hbm.at[src_v.at[ch + 2]],
                                     bufs[b + 2], gsems[b + 2])
                pltpu.async_copy(bufs[b], accs[p].at[dst_v.at[ch]],
                                 ssems[p], add=True)
            return 0
        lax.fori_loop(0, NLOOP, body, 0)
        pltpu.make_async_copy(bufs[2], accs[0].at[dst_v.at[NCHUNK - 2]],
                              ssems[0]).wait()
        pltpu.make_async_copy(bufs[3], accs[1].at[dst_v.at[NCHUNK - 1]],
                              ssems[1]).wait()

        plsc.subcore_barrier()
        pltpu.sync_copy(accB.at[pl.ds(s * NPS, NPS)], tbuf)
        for kk in range(NPS // 128):
            pltpu.sync_copy(tbuf.at[pl.ds(kk * 128, 128)], accA.at[iidx.at[kk]],
                            add=True)
        pltpu.sync_copy(accA.at[pl.ds(s * NPS, NPS)],
                        out_hbm.at[pl.ds(c * NN + s * NPS, NPS)])

    return msg_kernel


def _make_deg_kernel():
    @functools.partial(
        pl.kernel,
        out_type=jax.ShapeDtypeStruct((NC * NN, 32), jnp.float32),
        mesh=_mesh,
        compiler_params=_sc_params,
        scratch_types=[
            pltpu.VMEM((NCHUNK, K), jnp.int32),
            pltpu.VMEM((K, 32), jnp.float32),
            pltpu.VMEM((ZR, 32), jnp.float32),
            pltpu.VMEM_SHARED((NN, 32), jnp.float32),
        ],
    )
    def deg_kernel(eidx_hbm, out32_hbm, dst_v, ones32_v, zq32, acc32):
        c = lax.axis_index("c")
        s = lax.axis_index("s")
        wid = c * NS + s
        pltpu.sync_copy(eidx_hbm.at[NW + wid], dst_v)

        def fill(i, _):
            for j in range(2):
                ones32_v[i, pl.ds(j * 16, 16)] = jnp.ones((16,), jnp.float32)
            return 0
        lax.fori_loop(0, K, fill, 0)
        _zero_fill(zq32, ZR, 32)
        _zero_shared_slice(acc32, zq32, s * NPS)
        plsc.subcore_barrier()

        def body(i, _):
            pltpu.sync_copy(ones32_v, acc32.at[dst_v.at[i]], add=True)
            return 0
        lax.fori_loop(0, NCHUNK, body, 0)

        plsc.subcore_barrier()
        pltpu.sync_copy(acc32.at[pl.ds(s * NPS, NPS)],
                        out32_hbm.at[pl.ds(c * NN + s * NPS, NPS)])

    return deg_kernel


_deg_call = _make_deg_kernel()
_msg_call = {16: _make_msg_kernel(16), 32: _make_msg_kernel(32)}


G = 10
B4 = R4 // G
B8 = R8 // G


def _full_spec(shape):
    return pl.BlockSpec(shape, lambda i: tuple(0 for _ in shape))


def _blk(rows):
    return pl.BlockSpec((rows, 128), lambda i: (i, 0))


def _dual(rows):
    return (pl.BlockSpec((rows, 128), lambda i: (i, 0)),
            pl.BlockSpec((rows, 128), lambda i: (i + G, 0)))


def _dinv_of(deg):
    return jnp.where(deg > 0.0, 1.0 / jnp.sqrt(jnp.maximum(deg, 1e-12)), 0.0)


def _t0_body(xr_ref, w_ref, d4a_ref, d4b_ref, d8a_ref, d8b_ref,
             hw_ref, g_ref, dinv4_ref, dinv8_ref):
    dinv4 = _dinv_of(d4a_ref[...] + d4b_ref[...])
    dinv8 = _dinv_of(d8a_ref[...] + d8b_ref[...])
    hw = jnp.dot(xr_ref[...], w_ref[...], preferred_element_type=jnp.float32)
    hw_ref[...] = hw
    g_ref[...] = hw * dinv4
    dinv4_ref[...] = dinv4
    dinv8_ref[...] = dinv8


def _t0(xr, w0s, deg4, deg16):
    d4a, d4b = _dual(B4)
    d8a, d8b = _dual(B8)
    return pl.pallas_call(
        _t0_body,
        grid=(G,),
        in_specs=[pl.BlockSpec((B4, 512), lambda i: (i, 0)),
                  _full_spec(w0s.shape), d4a, d4b, d8a, d8b],
        out_specs=[_blk(B4), _blk(B4), _blk(B4), _blk(B8)],
        out_shape=[jax.ShapeDtypeStruct((R4, 128), jnp.float32),
                   jax.ShapeDtypeStruct((R4, 128), jnp.float32),
                   jax.ShapeDtypeStruct((R4, 128), jnp.float32),
                   jax.ShapeDtypeStruct((R8, 128), jnp.float32)],
    )(xr, w0s, deg4, deg4, deg16, deg16)


def _tl_body(aa_ref, ab_ref, hwp_ref, b_ref, dinv_ref, w_ref, hw_ref, g_ref):
    dinv = dinv_ref[...]
    h = jnp.tanh(DELTA * dinv * (aa_ref[...] + ab_ref[...])
                 + (1.0 - DELTA) * hwp_ref[...] + b_ref[...])
    hw = jnp.dot(h, w_ref[...], preferred_element_type=jnp.float32)
    hw_ref[...] = hw
    g_ref[...] = hw * dinv


def _tl(accp, hwp, bpk, dinvp, wbd, rows):
    aa, ab = _dual(rows)
    return pl.pallas_call(
        _tl_body,
        grid=(G,),
        in_specs=[aa, ab, _blk(rows), _full_spec(bpk.shape), _blk(rows),
                  _full_spec(wbd.shape)],
        out_specs=[_blk(rows), _blk(rows)],
        out_shape=[jax.ShapeDtypeStruct((rows * G, 128), jnp.float32),
                   jax.ShapeDtypeStruct((rows * G, 128), jnp.float32)],
    )(accp, accp, hwp, bpk, dinvp, wbd)


def _tf_body(aa_ref, ab_ref, hwp_ref, b_ref, dinv_ref, mw0_ref, mb0_ref,
             mw1_ref, mb1_ref, fw_ref, fb_ref, out_ref):
    dinv = dinv_ref[...]
    h = jnp.tanh(DELTA * dinv * (aa_ref[...] + ab_ref[...])
                 + (1.0 - DELTA) * hwp_ref[...] + b_ref[...])
    h = jax.nn.relu(jnp.dot(h, mw0_ref[...],
                            preferred_element_type=jnp.float32) + mb0_ref[...])
    h = jax.nn.relu(jnp.dot(h, mw1_ref[...],
                            preferred_element_type=jnp.float32) + mb1_ref[...])
    out_ref[...] = jnp.tanh(jnp.dot(h, fw_ref[...],
                                    preferred_element_type=jnp.float32)
                            + fb_ref[...])


def _tf(accp, hwp, bpk, dinvp, mw0bd, mb0pk, mw1bd, mb1pk, fwbd, fbpk):
    aa, ab = _dual(B8)
    return pl.pallas_call(
        _tf_body,
        grid=(G,),
        in_specs=[aa, ab, _blk(B8), _full_spec(bpk.shape), _blk(B8),
                  _full_spec(mw0bd.shape), _full_spec(mb0pk.shape),
                  _full_spec(mw1bd.shape), _full_spec(mb1pk.shape),
                  _full_spec(fwbd.shape), _full_spec(fbpk.shape)],
        out_specs=pl.BlockSpec((B8, 8), lambda i: (i, 0)),
        out_shape=jax.ShapeDtypeStruct((R8, 8), jnp.float32),
    )(accp, accp, hwp, bpk, dinvp, mw0bd, mb0pk, mw1bd, mb1pk, fwbd, fbpk)


def _pad2(a, r, c):
    return jnp.zeros((r, c), a.dtype).at[:a.shape[0], :a.shape[1]].set(a)


def kernel(x, edge_index, W0, b0, W1, b1, W2, b2, W3, b3, W4, b4, W5, b5,
           W6, b6, W7, b7, W8, b8, W9, b9, W10, b10, W11, b11,
           mW0, mb0, mW1, mb1, fW, fb):
    Ws = [W0, W1, W2, W3, W4, W5, W6, W7, W8, W9, W10, W11]
    bs = [b0, b1, b2, b3, b4, b5, b6, b7, b8, b9, b10, b11]
    f32 = jnp.float32

    pw = [max(16, w.shape[1]) for w in Ws]
    pin = [x.shape[1]] + pw[:-1]
    reps = [4 if w == 32 else 8 for w in pw]
    Wbd = []
    for l in range(12):
        wp = _pad2(Ws[l], pin[l], pw[l])
        k = 4 if l == 0 else reps[l - 1]
        if l > 0 and pin[l] != pw[l]:
            wp = _pad2(wp, pin[l], pin[l])
        Wbd.append(jnp.kron(jnp.eye(k, dtype=f32), wp))
    bpk = []
    for l in range(12):
        b_l = _pad2(bs[l].reshape(1, -1), 1, pw[l])
        bpk.append(jnp.tile(b_l, (1, 128 // b_l.shape[1])))
    mW0bd = jnp.kron(jnp.eye(8, dtype=f32), _pad2(mW0, 16, 16))
    mW1bd = jnp.kron(jnp.eye(8, dtype=f32), mW1)
    fWbd = jnp.kron(jnp.eye(8, dtype=f32), fW)
    mb0pk = jnp.tile(mb0.reshape(1, -1), (1, 8))
    mb1pk = jnp.tile(mb1.reshape(1, -1), (1, 8))
    fbpk = jnp.tile(fb.reshape(1, -1), (1, 8))

    eidx = edge_index.reshape(2 * NW, NCHUNK, K)
    xr = _pad2(x, NN, 128).reshape(R4, 512)

    deg32 = _deg_call(eidx)
    deg4p = deg32.reshape(2 * R4, 128)
    deg8p = deg32[:, :16].reshape(2 * R8, 128)

    hw, g, dinv4, dinv8 = _t0(xr, Wbd[0], deg4p, deg8p)
    for l in range(12):
        width = pw[l]
        rep = reps[l]
        rows = R4 if rep == 4 else R8
        if l > 0 and pw[l - 1] != width:
            g_lin = g.reshape(NN, 32)[:, :16]
        else:
            g_lin = g.reshape(NN, width)
        acc = _msg_call[width](g_lin, eidx)
        accp = acc.reshape(2 * rows, 128)
        if l > 0 and pw[l - 1] != width:
            hwp = hw.reshape(NN, 32)[:, :16].reshape(R8, 128)
        else:
            hwp = hw
        dinvp = dinv4 if rep == 4 else dinv8
        if l < 11:
            hw, g = _tl(accp, hwp, bpk[l], dinvp, Wbd[l + 1], rows // G)
        else:
            outp = _tf(accp, hwp, bpk[l], dinvp, mW0bd, mb0pk, mW1bd,
                       mb1pk, fWbd, fbpk)
    return outp.reshape(NN, 1)[:N]


# speedup vs baseline: 1.5009x; 1.5009x over previous
import functools

import jax
import jax.numpy as jnp
from jax import lax
from jax.experimental import pallas as pl
from jax.experimental.pallas import tpu as pltpu
from jax.experimental.pallas import tpu_sc as plsc

N = 10000
E = 320000
DELTA = 2.0

NC = 2
NS = 16
NW = NC * NS
EPT = E // NW
K = 125
NCHUNK = EPT // K
NBUF = 8
NN = 10240
NPS = NN // NS
ZR = 80

R4 = NN * 32 // 128
R8 = NN * 16 // 128

_mesh = plsc.VectorSubcoreMesh(core_axis_name="c", subcore_axis_name="s")
_sc_params = pltpu.CompilerParams(use_tc_tiling_on_sc=False)


def _zero_fill(ref, rows, width):
    def body(i, _):
        for j in range(width // 16):
            ref[i, pl.ds(j * 16, 16)] = jnp.zeros((16,), jnp.float32)
        return 0
    lax.fori_loop(0, rows, body, 0)


def _zero_shared_slice(acc, zq, row0):
    for r in range(NPS // ZR):
        pltpu.sync_copy(zq, acc.at[pl.ds(row0 + r * ZR, ZR)])


def _make_msg_kernel(width):
    @functools.partial(
        pl.kernel,
        out_type=jax.ShapeDtypeStruct((NC * NN, width), jnp.float32),
        mesh=_mesh,
        compiler_params=_sc_params,
        scratch_types=[
            pltpu.VMEM((NCHUNK, K), jnp.int32),
            pltpu.VMEM((NCHUNK, K), jnp.int32),
            [pltpu.VMEM((K, width), jnp.float32)] * NBUF,
            pltpu.VMEM((ZR, width), jnp.float32),
            pltpu.VMEM_SHARED((NN, width), jnp.float32),
            [pltpu.SemaphoreType.DMA] * NBUF,
        ],
    )
    def msg_kernel(g_hbm, eidx_hbm, out_hbm, src_v, dst_v, bufs, zq,
                   acc, gsems):
        c = lax.axis_index("c")
        s = lax.axis_index("s")
        wid = c * NS + s
        pltpu.sync_copy(eidx_hbm.at[wid], src_v)
        pltpu.sync_copy(eidx_hbm.at[NW + wid], dst_v)
        _zero_fill(zq, ZR, width)
        _zero_shared_slice(acc, zq, s * NPS)
        plsc.subcore_barrier()

        NLOOP = NCHUNK // NBUF
        for b in range(NBUF):
            pltpu.async_copy(g_hbm.at[src_v.at[b]], bufs[b], gsems[b])

        def body(i, _):
            base = i * NBUF
            for b in range(NBUF):
                pltpu.make_async_copy(g_hbm.at[src_v.at[base + b]], bufs[b],
                                      gsems[b]).wait()
                pltpu.sync_copy(bufs[b], acc.at[dst_v.at[base + b]],
                                add=True)
                @pl.when(i + 1 < NLOOP)
                def _():
                    pltpu.async_copy(g_hbm.at[src_v.at[base + NBUF + b]],
                                     bufs[b], gsems[b])
            return 0
        lax.fori_loop(0, NLOOP, body, 0)

        plsc.subcore_barrier()
        pltpu.sync_copy(acc.at[pl.ds(s * NPS, NPS)],
                        out_hbm.at[pl.ds(c * NN + s * NPS, NPS)])

    return msg_kernel


def _make_deg_kernel():
    @functools.partial(
        pl.kernel,
        out_type=jax.ShapeDtypeStruct((NC * NN, 32), jnp.float32),
        mesh=_mesh,
        compiler_params=_sc_params,
        scratch_types=[
            pltpu.VMEM((NCHUNK, K), jnp.int32),
            pltpu.VMEM((K, 32), jnp.float32),
            pltpu.VMEM((ZR, 32), jnp.float32),
            pltpu.VMEM_SHARED((NN, 32), jnp.float32),
        ],
    )
    def deg_kernel(eidx_hbm, out32_hbm, dst_v, ones32_v, zq32, acc32):
        c = lax.axis_index("c")
        s = lax.axis_index("s")
        wid = c * NS + s
        pltpu.sync_copy(eidx_hbm.at[NW + wid], dst_v)

        def fill(i, _):
            for j in range(2):
                ones32_v[i, pl.ds(j * 16, 16)] = jnp.ones((16,), jnp.float32)
            return 0
        lax.fori_loop(0, K, fill, 0)
        _zero_fill(zq32, ZR, 32)
        _zero_shared_slice(acc32, zq32, s * NPS)
        plsc.subcore_barrier()

        def body(i, _):
            pltpu.sync_copy(ones32_v, acc32.at[dst_v.at[i]], add=True)
            return 0
        lax.fori_loop(0, NCHUNK, body, 0)

        plsc.subcore_barrier()
        pltpu.sync_copy(acc32.at[pl.ds(s * NPS, NPS)],
                        out32_hbm.at[pl.ds(c * NN + s * NPS, NPS)])

    return deg_kernel


_deg_call = _make_deg_kernel()
_msg_call = {16: _make_msg_kernel(16), 32: _make_msg_kernel(32)}


G = 10
B4 = R4 // G
B8 = R8 // G


def _full_spec(shape):
    return pl.BlockSpec(shape, lambda i: tuple(0 for _ in shape))


def _blk(rows):
    return pl.BlockSpec((rows, 128), lambda i: (i, 0))


def _dual(rows):
    return (pl.BlockSpec((rows, 128), lambda i: (i, 0)),
            pl.BlockSpec((rows, 128), lambda i: (i + G, 0)))


def _dinv_of(deg):
    return jnp.where(deg > 0.0, 1.0 / jnp.sqrt(jnp.maximum(deg, 1e-12)), 0.0)


def _t0_body(xr_ref, w_ref, d4a_ref, d4b_ref, d8a_ref, d8b_ref,
             hw_ref, g_ref, dinv4_ref, dinv8_ref):
    dinv4 = _dinv_of(d4a_ref[...] + d4b_ref[...])
    dinv8 = _dinv_of(d8a_ref[...] + d8b_ref[...])
    hw = jnp.dot(xr_ref[...], w_ref[...], preferred_element_type=jnp.float32)
    hw_ref[...] = hw
    g_ref[...] = hw * dinv4
    dinv4_ref[...] = dinv4
    dinv8_ref[...] = dinv8


def _t0(xr, w0s, deg4, deg16):
    d4a, d4b = _dual(B4)
    d8a, d8b = _dual(B8)
    return pl.pallas_call(
        _t0_body,
        grid=(G,),
        in_specs=[pl.BlockSpec((B4, 512), lambda i: (i, 0)),
                  _full_spec(w0s.shape), d4a, d4b, d8a, d8b],
        out_specs=[_blk(B4), _blk(B4), _blk(B4), _blk(B8)],
        out_shape=[jax.ShapeDtypeStruct((R4, 128), jnp.float32),
                   jax.ShapeDtypeStruct((R4, 128), jnp.float32),
                   jax.ShapeDtypeStruct((R4, 128), jnp.float32),
                   jax.ShapeDtypeStruct((R8, 128), jnp.float32)],
    )(xr, w0s, deg4, deg4, deg16, deg16)


def _tl_body(aa_ref, ab_ref, hwp_ref, b_ref, dinv_ref, w_ref, hw_ref, g_ref):
    dinv = dinv_ref[...]
    h = jnp.tanh(DELTA * dinv * (aa_ref[...] + ab_ref[...])
                 + (1.0 - DELTA) * hwp_ref[...] + b_ref[...])
    hw = jnp.dot(h, w_ref[...], preferred_element_type=jnp.float32)
    hw_ref[...] = hw
    g_ref[...] = hw * dinv


def _tl(accp, hwp, bpk, dinvp, wbd, rows):
    aa, ab = _dual(rows)
    return pl.pallas_call(
        _tl_body,
        grid=(G,),
        in_specs=[aa, ab, _blk(rows), _full_spec(bpk.shape), _blk(rows),
                  _full_spec(wbd.shape)],
        out_specs=[_blk(rows), _blk(rows)],
        out_shape=[jax.ShapeDtypeStruct((rows * G, 128), jnp.float32),
                   jax.ShapeDtypeStruct((rows * G, 128), jnp.float32)],
    )(accp, accp, hwp, bpk, dinvp, wbd)


def _tf_body(aa_ref, ab_ref, hwp_ref, b_ref, dinv_ref, mw0_ref, mb0_ref,
             mw1_ref, mb1_ref, fw_ref, fb_ref, out_ref):
    dinv = dinv_ref[...]
    h = jnp.tanh(DELTA * dinv * (aa_ref[...] + ab_ref[...])
                 + (1.0 - DELTA) * hwp_ref[...] + b_ref[...])
    h = jax.nn.relu(jnp.dot(h, mw0_ref[...],
                            preferred_element_type=jnp.float32) + mb0_ref[...])
    h = jax.nn.relu(jnp.dot(h, mw1_ref[...],
                            preferred_element_type=jnp.float32) + mb1_ref[...])
    out_ref[...] = jnp.tanh(jnp.dot(h, fw_ref[...],
                                    preferred_element_type=jnp.float32)
                            + fb_ref[...])


def _tf(accp, hwp, bpk, dinvp, mw0bd, mb0pk, mw1bd, mb1pk, fwbd, fbpk):
    aa, ab = _dual(B8)
    return pl.pallas_call(
        _tf_body,
        grid=(G,),
        in_specs=[aa, ab, _blk(B8), _full_spec(bpk.shape), _blk(B8),
                  _full_spec(mw0bd.shape), _full_spec(mb0pk.shape),
                  _full_spec(mw1bd.shape), _full_spec(mb1pk.shape),
                  _full_spec(fwbd.shape), _full_spec(fbpk.shape)],
        out_specs=pl.BlockSpec((B8, 8), lambda i: (i, 0)),
        out_shape=jax.ShapeDtypeStruct((R8, 8), jnp.float32),
    )(accp, accp, hwp, bpk, dinvp, mw0bd, mb0pk, mw1bd, mb1pk, fwbd, fbpk)


def _pad2(a, r, c):
    return jnp.zeros((r, c), a.dtype).at[:a.shape[0], :a.shape[1]].set(a)


def kernel(x, edge_index, W0, b0, W1, b1, W2, b2, W3, b3, W4, b4, W5, b5,
           W6, b6, W7, b7, W8, b8, W9, b9, W10, b10, W11, b11,
           mW0, mb0, mW1, mb1, fW, fb):
    Ws = [W0, W1, W2, W3, W4, W5, W6, W7, W8, W9, W10, W11]
    bs = [b0, b1, b2, b3, b4, b5, b6, b7, b8, b9, b10, b11]
    f32 = jnp.float32

    pw = [max(16, w.shape[1]) for w in Ws]
    pin = [x.shape[1]] + pw[:-1]
    reps = [4 if w == 32 else 8 for w in pw]
    Wbd = []
    for l in range(12):
        wp = _pad2(Ws[l], pin[l], pw[l])
        k = 4 if l == 0 else reps[l - 1]
        if l > 0 and pin[l] != pw[l]:
            wp = _pad2(wp, pin[l], pin[l])
        Wbd.append(jnp.kron(jnp.eye(k, dtype=f32), wp))
    bpk = []
    for l in range(12):
        b_l = _pad2(bs[l].reshape(1, -1), 1, pw[l])
        bpk.append(jnp.tile(b_l, (1, 128 // b_l.shape[1])))
    mW0bd = jnp.kron(jnp.eye(8, dtype=f32), _pad2(mW0, 16, 16))
    mW1bd = jnp.kron(jnp.eye(8, dtype=f32), mW1)
    fWbd = jnp.kron(jnp.eye(8, dtype=f32), fW)
    mb0pk = jnp.tile(mb0.reshape(1, -1), (1, 8))
    mb1pk = jnp.tile(mb1.reshape(1, -1), (1, 8))
    fbpk = jnp.tile(fb.reshape(1, -1), (1, 8))

    eidx = edge_index.reshape(2 * NW, NCHUNK, K)
    xr = _pad2(x, NN, 128).reshape(R4, 512)

    deg32 = _deg_call(eidx)
    deg4p = deg32.reshape(2 * R4, 128)
    deg8p = deg32[:, :16].reshape(2 * R8, 128)

    hw, g, dinv4, dinv8 = _t0(xr, Wbd[0], deg4p, deg8p)
    for l in range(12):
        width = pw[l]
        rep = reps[l]
        rows = R4 if rep == 4 else R8
        if l > 0 and pw[l - 1] != width:
            g_lin = g.reshape(NN, 32)[:, :16]
        else:
            g_lin = g.reshape(NN, width)
        acc = _msg_call[width](g_lin, eidx)
        accp = acc.reshape(2 * rows, 128)
        if l > 0 and pw[l - 1] != width:
            hwp = hw.reshape(NN, 32)[:, :16].reshape(R8, 128)
        else:
            hwp = hw
        dinvp = dinv4 if rep == 4 else dinv8
        if l < 11:
            hw, g = _tl(accp, hwp, bpk[l], dinvp, Wbd[l + 1], rows // G)
        else:
            outp = _tf(accp, hwp, bpk[l], dinvp, mW0bd, mb0pk, mW1bd,
                       mb1pk, fWbd, fbpk)
    return outp.reshape(NN, 1)[:N]


# speedup vs baseline: 1.5371x; 1.0241x over previous
import functools

import jax
import jax.numpy as jnp
from jax import lax
from jax.experimental import pallas as pl
from jax.experimental.pallas import tpu as pltpu
from jax.experimental.pallas import tpu_sc as plsc

N = 10000
E = 320000
DELTA = 2.0

NC = 2
NS = 16
NW = NC * NS
EPT = E // NW
K = 125
NCHUNK = EPT // K
NBUF = 8
NN = 10240
NPS = NN // NS
ZR = 80

R4 = NN * 32 // 128
R8 = NN * 16 // 128

_mesh = plsc.VectorSubcoreMesh(core_axis_name="c", subcore_axis_name="s")
_sc_params = pltpu.CompilerParams(use_tc_tiling_on_sc=False)


def _zero_fill(ref, rows, width):
    def body(i, _):
        for j in range(width // 16):
            ref[i, pl.ds(j * 16, 16)] = jnp.zeros((16,), jnp.float32)
        return 0
    lax.fori_loop(0, rows, body, 0)


def _zero_shared_slice(acc, zq, row0):
    for r in range(NPS // ZR):
        pltpu.sync_copy(zq, acc.at[pl.ds(row0 + r * ZR, ZR)])


def _make_msg_kernel(width):
    @functools.partial(
        pl.kernel,
        out_type=jax.ShapeDtypeStruct((NC * NN, width), jnp.float32),
        mesh=_mesh,
        compiler_params=_sc_params,
        scratch_types=[
            pltpu.VMEM((NCHUNK, K), jnp.int32),
            pltpu.VMEM((NCHUNK, K), jnp.int32),
            [pltpu.VMEM((K, width), jnp.float32)] * NBUF,
            pltpu.VMEM((ZR, width), jnp.float32),
            pltpu.VMEM_SHARED((NN, width), jnp.float32),
            [pltpu.SemaphoreType.DMA] * NBUF,
        ],
    )
    def msg_kernel(g_hbm, eidx_hbm, out_hbm, src_v, dst_v, bufs, zq,
                   acc, gsems):
        c = lax.axis_index("c")
        s = lax.axis_index("s")
        wid = c * NS + s
        pltpu.sync_copy(eidx_hbm.at[wid], src_v)
        pltpu.sync_copy(eidx_hbm.at[NW + wid], dst_v)

        NLOOP = NCHUNK // NBUF
        for b in range(NBUF):
            pltpu.async_copy(g_hbm.at[src_v.at[b]], bufs[b], gsems[b])

        _zero_fill(zq, ZR, width)
        _zero_shared_slice(acc, zq, s * NPS)
        plsc.subcore_barrier()

        def body(i, _):
            base = i * NBUF
            for b in range(NBUF):
                pltpu.make_async_copy(g_hbm.at[src_v.at[base + b]], bufs[b],
                                      gsems[b]).wait()
                pltpu.sync_copy(bufs[b], acc.at[dst_v.at[base + b]],
                                add=True)
                @pl.when(i + 1 < NLOOP)
                def _():
                    pltpu.async_copy(g_hbm.at[src_v.at[base + NBUF + b]],
                                     bufs[b], gsems[b])
            return 0
        lax.fori_loop(0, NLOOP, body, 0)

        plsc.subcore_barrier()
        pltpu.sync_copy(acc.at[pl.ds(s * NPS, NPS)],
                        out_hbm.at[pl.ds(c * NN + s * NPS, NPS)])

    return msg_kernel


def _make_deg_kernel():
    @functools.partial(
        pl.kernel,
        out_type=jax.ShapeDtypeStruct((NC * NN, 32), jnp.float32),
        mesh=_mesh,
        compiler_params=_sc_params,
        scratch_types=[
            pltpu.VMEM((NCHUNK, K), jnp.int32),
            pltpu.VMEM((K, 32), jnp.float32),
            pltpu.VMEM((ZR, 32), jnp.float32),
            pltpu.VMEM_SHARED((NN, 32), jnp.float32),
        ],
    )
    def deg_kernel(eidx_hbm, out32_hbm, dst_v, ones32_v, zq32, acc32):
        c = lax.axis_index("c")
        s = lax.axis_index("s")
        wid = c * NS + s
        pltpu.sync_copy(eidx_hbm.at[NW + wid], dst_v)

        def fill(i, _):
            for j in range(2):
                ones32_v[i, pl.ds(j * 16, 16)] = jnp.ones((16,), jnp.float32)
            return 0
        lax.fori_loop(0, K, fill, 0)
        _zero_fill(zq32, ZR, 32)
        _zero_shared_slice(acc32, zq32, s * NPS)
        plsc.subcore_barrier()

        def body(i, _):
            pltpu.sync_copy(ones32_v, acc32.at[dst_v.at[i]], add=True)
            return 0
        lax.fori_loop(0, NCHUNK, body, 0)

        plsc.subcore_barrier()
        pltpu.sync_copy(acc32.at[pl.ds(s * NPS, NPS)],
                        out32_hbm.at[pl.ds(c * NN + s * NPS, NPS)])

    return deg_kernel


_deg_call = _make_deg_kernel()
_msg_call = {16: _make_msg_kernel(16), 32: _make_msg_kernel(32)}


G = 10
B4 = R4 // G
B8 = R8 // G


def _full_spec(shape):
    return pl.BlockSpec(shape, lambda i: tuple(0 for _ in shape))


def _blk(rows):
    return pl.BlockSpec((rows, 128), lambda i: (i, 0))


def _dual(rows):
    return (pl.BlockSpec((rows, 128), lambda i: (i, 0)),
            pl.BlockSpec((rows, 128), lambda i: (i + G, 0)))


def _dinv_of(deg):
    return jnp.where(deg > 0.0, 1.0 / jnp.sqrt(jnp.maximum(deg, 1e-12)), 0.0)


def _t0_body(xr_ref, w_ref, d4a_ref, d4b_ref, d8a_ref, d8b_ref,
             hw_ref, g_ref, dinv4_ref, dinv8_ref):
    dinv4 = _dinv_of(d4a_ref[...] + d4b_ref[...])
    dinv8 = _dinv_of(d8a_ref[...] + d8b_ref[...])
    hw = jnp.dot(xr_ref[...], w_ref[...], preferred_element_type=jnp.float32)
    hw_ref[...] = hw
    g_ref[...] = hw * dinv4
    dinv4_ref[...] = dinv4
    dinv8_ref[...] = dinv8


def _t0(xr, w0s, deg4, deg16):
    d4a, d4b = _dual(B4)
    d8a, d8b = _dual(B8)
    return pl.pallas_call(
        _t0_body,
        grid=(G,),
        in_specs=[pl.BlockSpec((B4, 512), lambda i: (i, 0)),
                  _full_spec(w0s.shape), d4a, d4b, d8a, d8b],
        out_specs=[_blk(B4), _blk(B4), _blk(B4), _blk(B8)],
        out_shape=[jax.ShapeDtypeStruct((R4, 128), jnp.float32),
                   jax.ShapeDtypeStruct((R4, 128), jnp.float32),
                   jax.ShapeDtypeStruct((R4, 128), jnp.float32),
                   jax.ShapeDtypeStruct((R8, 128), jnp.float32)],
    )(xr, w0s, deg4, deg4, deg16, deg16)


def _tl_body(aa_ref, ab_ref, hwp_ref, b_ref, dinv_ref, w_ref, hw_ref, g_ref):
    dinv = dinv_ref[...]
    h = jnp.tanh(DELTA * dinv * (aa_ref[...] + ab_ref[...])
                 + (1.0 - DELTA) * hwp_ref[...] + b_ref[...])
    hw = jnp.dot(h, w_ref[...], preferred_element_type=jnp.float32)
    hw_ref[...] = hw
    g_ref[...] = hw * dinv


def _tl(accp, hwp, bpk, dinvp, wbd, rows):
    aa, ab = _dual(rows)
    return pl.pallas_call(
        _tl_body,
        grid=(G,),
        in_specs=[aa, ab, _blk(rows), _full_spec(bpk.shape), _blk(rows),
                  _full_spec(wbd.shape)],
        out_specs=[_blk(rows), _blk(rows)],
        out_shape=[jax.ShapeDtypeStruct((rows * G, 128), jnp.float32),
                   jax.ShapeDtypeStruct((rows * G, 128), jnp.float32)],
    )(accp, accp, hwp, bpk, dinvp, wbd)


def _tf_body(aa_ref, ab_ref, hwp_ref, b_ref, dinv_ref, mw0_ref, mb0_ref,
             mw1_ref, mb1_ref, fw_ref, fb_ref, out_ref):
    dinv = dinv_ref[...]
    h = jnp.tanh(DELTA * dinv * (aa_ref[...] + ab_ref[...])
                 + (1.0 - DELTA) * hwp_ref[...] + b_ref[...])
    h = jax.nn.relu(jnp.dot(h, mw0_ref[...],
                            preferred_element_type=jnp.float32) + mb0_ref[...])
    h = jax.nn.relu(jnp.dot(h, mw1_ref[...],
                            preferred_element_type=jnp.float32) + mb1_ref[...])
    out_ref[...] = jnp.tanh(jnp.dot(h, fw_ref[...],
                                    preferred_element_type=jnp.float32)
                            + fb_ref[...])


def _tf(accp, hwp, bpk, dinvp, mw0bd, mb0pk, mw1bd, mb1pk, fwbd, fbpk):
    aa, ab = _dual(B8)
    return pl.pallas_call(
        _tf_body,
        grid=(G,),
        in_specs=[aa, ab, _blk(B8), _full_spec(bpk.shape), _blk(B8),
                  _full_spec(mw0bd.shape), _full_spec(mb0pk.shape),
                  _full_spec(mw1bd.shape), _full_spec(mb1pk.shape),
                  _full_spec(fwbd.shape), _full_spec(fbpk.shape)],
        out_specs=pl.BlockSpec((B8, 8), lambda i: (i, 0)),
        out_shape=jax.ShapeDtypeStruct((R8, 8), jnp.float32),
    )(accp, accp, hwp, bpk, dinvp, mw0bd, mb0pk, mw1bd, mb1pk, fwbd, fbpk)


def _pad2(a, r, c):
    return jnp.zeros((r, c), a.dtype).at[:a.shape[0], :a.shape[1]].set(a)


def kernel(x, edge_index, W0, b0, W1, b1, W2, b2, W3, b3, W4, b4, W5, b5,
           W6, b6, W7, b7, W8, b8, W9, b9, W10, b10, W11, b11,
           mW0, mb0, mW1, mb1, fW, fb):
    Ws = [W0, W1, W2, W3, W4, W5, W6, W7, W8, W9, W10, W11]
    bs = [b0, b1, b2, b3, b4, b5, b6, b7, b8, b9, b10, b11]
    f32 = jnp.float32

    pw = [max(16, w.shape[1]) for w in Ws]
    pin = [x.shape[1]] + pw[:-1]
    reps = [4 if w == 32 else 8 for w in pw]
    Wbd = []
    for l in range(12):
        wp = _pad2(Ws[l], pin[l], pw[l])
        k = 4 if l == 0 else reps[l - 1]
        if l > 0 and pin[l] != pw[l]:
            wp = _pad2(wp, pin[l], pin[l])
        Wbd.append(jnp.kron(jnp.eye(k, dtype=f32), wp))
    bpk = []
    for l in range(12):
        b_l = _pad2(bs[l].reshape(1, -1), 1, pw[l])
        bpk.append(jnp.tile(b_l, (1, 128 // b_l.shape[1])))
    mW0bd = jnp.kron(jnp.eye(8, dtype=f32), _pad2(mW0, 16, 16))
    mW1bd = jnp.kron(jnp.eye(8, dtype=f32), mW1)
    fWbd = jnp.kron(jnp.eye(8, dtype=f32), fW)
    mb0pk = jnp.tile(mb0.reshape(1, -1), (1, 8))
    mb1pk = jnp.tile(mb1.reshape(1, -1), (1, 8))
    fbpk = jnp.tile(fb.reshape(1, -1), (1, 8))

    eidx = edge_index.reshape(2 * NW, NCHUNK, K)
    xr = _pad2(x, NN, 128).reshape(R4, 512)

    deg32 = _deg_call(eidx)
    deg4p = deg32.reshape(2 * R4, 128)
    deg8p = jnp.broadcast_to(deg32[:, :1], (2 * NN, 16)).reshape(2 * R8, 128)

    hw, g, dinv4, dinv8 = _t0(xr, Wbd[0], deg4p, deg8p)
    for l in range(12):
        width = pw[l]
        rep = reps[l]
        rows = R4 if rep == 4 else R8
        if l > 0 and pw[l - 1] != width:
            g_lin = g.reshape(NN, 32)[:, :16]
        else:
            g_lin = g.reshape(NN, width)
        acc = _msg_call[width](g_lin, eidx)
        accp = acc.reshape(2 * rows, 128)
        if l > 0 and pw[l - 1] != width:
            hwp = hw.reshape(NN, 32)[:, :16].reshape(R8, 128)
        else:
            hwp = hw
        dinvp = dinv4 if rep == 4 else dinv8
        if l < 11:
            hw, g = _tl(accp, hwp, bpk[l], dinvp, Wbd[l + 1], rows // G)
        else:
            outp = _tf(accp, hwp, bpk[l], dinvp, mW0bd, mb0pk, mW1bd,
                       mb1pk, fWbd, fbpk)
    return outp.reshape(NN, 1)[:N]
